# baseline (speedup 1.0000x reference)
import jax
import jax.numpy as jnp
from jax.experimental import pallas as pl
from jax.experimental.pallas import tpu as pltpu

_H1, _W1 = 75, 69
_HP, _WP = 77, 80
_WC = 72
_P1H, _P1W = 39, 40
_WC2 = 36
_C1, _C2, _NCLS = 16, 4, 2
_H3, _W3 = 18, 17
_LEAK = 0.01
_BL = 128


def _fwd_kernel(x_ref, w1_ref, b1_ref, w2_ref, b2_ref, wdb_ref, bd_ref,
                out_ref, p1a_ref, p1b_ref, xs_ref, pwb_ref):
    @pl.when(pl.program_id(0) == 0)
    def _init():
        p1a_ref[...] = jnp.zeros_like(p1a_ref)
        p1b_ref[...] = jnp.zeros_like(p1b_ref)

    for kx in range(3):
        xs_ref[kx] = x_ref[:, kx:kx + _WC, :]

    def pool_cols(mv, w, win):
        pwb_ref[0:1, 0:w, :] = mv
        ev = pwb_ref[0:1, pl.ds(0, win // 2, 2), :]
        od = pwb_ref[0:1, pl.ds(1, win // 2, 2), :]
        return jnp.maximum(ev, od)

    def c1_body(rp, carry):
        r0 = 2 * rp
        for g in range(_C1 // 4):
            c0 = g * 4
            accs = [jnp.zeros((2, _WC, _BL), jnp.float32) for _ in range(4)]
            for ky in range(3):
                for kx in range(3):
                    k = ky * 3 + kx
                    p = xs_ref[kx, pl.ds(r0 + ky, 2), :, :]
                    for cc in range(4):
                        accs[cc] = accs[cc] + w1_ref[k * _C1 + c0 + cc] * p
            for cc, a in enumerate(accs):
                mv = jnp.maximum(a[0:1], a[1:2])
                pw = pool_cols(mv, _WC, 68)
                h = pw + b1_ref[c0 + cc]
                h = jnp.maximum(h, _LEAK * h)
                p1a_ref[c0 + cc, pl.ds(1 + rp, 1), 1:35, :] = h
                p1b_ref[c0 + cc, pl.ds(1 + rp, 1), 0:34, :] = h
        return carry
    jax.lax.fori_loop(0, 37, c1_body, 0)

    def c2_body(rp, oacc):
        oa0, oa1 = oacc
        z = jnp.zeros((2, _WC2, _BL), jnp.float32)
        accs = [z, z, z, z]
        for ci in range(_C1):
            for ky in range(3):
                for kx in range(3):
                    k = ky * 3 + kx
                    if kx == 0:
                        p = p1a_ref[ci, pl.ds(2 * rp + ky, 2), 0:_WC2, :]
                    elif kx == 1:
                        p = p1b_ref[ci, pl.ds(2 * rp + ky, 2), 0:_WC2, :]
                    else:
                        p = p1a_ref[ci, pl.ds(2 * rp + ky, 2), 2:2 + _WC2, :]
                    for co in range(_C2):
                        accs[co] = (accs[co]
                                    + w2_ref[(k * _C2 + co) * _C1 + ci] * p)
        for co in range(_C2):
            mv = jnp.maximum(accs[co][0:1], accs[co][1:2])
            pw = pool_cols(mv, _WC2, 34)
            h = pw + b2_ref[co]
            h = jnp.maximum(h, _LEAK * h)
            w0 = wdb_ref[0, co, pl.ds(rp, 1), :, :]
            w1 = wdb_ref[1, co, pl.ds(rp, 1), :, :]
            oa0 = oa0 + jnp.sum(w0 * h, axis=(0, 1))
            oa1 = oa1 + jnp.sum(w1 * h, axis=(0, 1))
        return oa0, oa1
    oa0 = jnp.full((_BL,), bd_ref[0], jnp.float32)
    oa1 = jnp.full((_BL,), bd_ref[1], jnp.float32)
    oa0, oa1 = jax.lax.fori_loop(0, 18, c2_body, (oa0, oa1))
    out_ref[...] = jnp.stack([oa0, oa1])


def kernel(x, w1k, b1, w2k, b2, wd4, bd, rd1, cd1, rd2, cd2):
    del rd1, cd1, rd2, cd2
    x = x.reshape(-1, _H1, _W1).astype(jnp.float32)
    B = x.shape[0]
    Bp = ((B + _BL - 1) // _BL) * _BL
    xp = jnp.pad(x, ((0, Bp - B), (1, 1), (1, _WP - _W1 - 1)))
    xt = jnp.transpose(xp, (1, 2, 0))
    wdb = jnp.broadcast_to(wd4[..., None],
                           (_NCLS, _C2, _H3, _W3, _BL)).astype(jnp.float32)

    def _smem():
        return pl.BlockSpec(memory_space=pltpu.MemorySpace.SMEM)

    out = pl.pallas_call(
        _fwd_kernel,
        out_shape=jax.ShapeDtypeStruct((_NCLS, Bp), jnp.float32),
        grid=(Bp // _BL,),
        in_specs=[
            pl.BlockSpec((_HP, _WP, _BL), lambda b: (0, 0, b)),
            _smem(),
            _smem(),
            _smem(),
            _smem(),
            pl.BlockSpec((_NCLS, _C2, _H3, _W3, _BL),
                         lambda b: (0, 0, 0, 0, 0)),
            _smem(),
        ],
        out_specs=pl.BlockSpec((_NCLS, _BL), lambda b: (0, b)),
        scratch_shapes=[
            pltpu.VMEM((_C1, _P1H, _P1W, _BL), jnp.float32),
            pltpu.VMEM((_C1, _P1H, _P1W, _BL), jnp.float32),
            pltpu.VMEM((3, _HP, _WC, _BL), jnp.float32),
            pltpu.VMEM((1, _WC, _BL), jnp.float32),
        ],
        compiler_params=pltpu.CompilerParams(
            dimension_semantics=("arbitrary",)),
    )(xt, w1k, b1, w2k, b2, wdb, bd)
    return jnp.transpose(out)[:B]


# speedup vs baseline: 1.0195x; 1.0195x over previous
import jax
import jax.numpy as jnp
from jax.experimental import pallas as pl
from jax.experimental.pallas import tpu as pltpu

_H1, _W1 = 75, 69
_HP, _WP = 77, 80
_WC = 72
_P1H, _P1W = 39, 40
_WC2 = 36
_C1, _C2, _NCLS = 16, 4, 2
_H3, _W3 = 18, 17
_LEAK = 0.01
_BL = 128


def _fwd_kernel(x_ref, w1_ref, b1_ref, w2_ref, b2_ref, wdb_ref, bd_ref,
                out_ref, p1a_ref, p1b_ref, xs_ref, pwb_ref):
    @pl.when(pl.program_id(0) == 0)
    def _init():
        p1a_ref[...] = jnp.zeros_like(p1a_ref)
        p1b_ref[...] = jnp.zeros_like(p1b_ref)

    for kx in range(3):
        xs_ref[kx] = x_ref[:, kx:kx + _WC, :]

    def pool_cols(mv, w, win):
        pwb_ref[0:1, 0:w, :] = mv
        ev = pwb_ref[0:1, pl.ds(0, win // 2, 2), :]
        od = pwb_ref[0:1, pl.ds(1, win // 2, 2), :]
        return jnp.maximum(ev, od)

    def c1_body(rp, carry):
        r0 = 2 * rp
        for g in range(_C1 // 2):
            c0 = g * 2
            accs = [jnp.zeros((2, _WC, _BL), jnp.float32) for _ in range(2)]
            for ky in range(3):
                for kx in range(3):
                    k = ky * 3 + kx
                    p = xs_ref[kx, pl.ds(r0 + ky, 2), :, :]
                    for cc in range(2):
                        accs[cc] = accs[cc] + w1_ref[k * _C1 + c0 + cc] * p
            for cc, a in enumerate(accs):
                mv = jnp.maximum(a[0:1], a[1:2])
                pw = pool_cols(mv, _WC, 68)
                h = pw + b1_ref[c0 + cc]
                h = jnp.maximum(h, _LEAK * h)
                p1a_ref[c0 + cc, pl.ds(1 + rp, 1), 1:35, :] = h
                p1b_ref[c0 + cc, pl.ds(1 + rp, 1), 0:34, :] = h
        return carry
    jax.lax.fori_loop(0, 37, c1_body, 0)

    def c2_body(rp, oacc):
        oa0, oa1 = oacc
        z = jnp.zeros((2, _WC2, _BL), jnp.float32)
        accs = [z, z, z, z]
        for ci in range(_C1):
            for ky in range(3):
                for kx in range(3):
                    k = ky * 3 + kx
                    if kx == 0:
                        p = p1a_ref[ci, pl.ds(2 * rp + ky, 2), 0:_WC2, :]
                    elif kx == 1:
                        p = p1b_ref[ci, pl.ds(2 * rp + ky, 2), 0:_WC2, :]
                    else:
                        p = p1a_ref[ci, pl.ds(2 * rp + ky, 2), 2:2 + _WC2, :]
                    for co in range(_C2):
                        accs[co] = (accs[co]
                                    + w2_ref[(k * _C2 + co) * _C1 + ci] * p)
        for co in range(_C2):
            mv = jnp.maximum(accs[co][0:1], accs[co][1:2])
            pw = pool_cols(mv, _WC2, 34)
            h = pw + b2_ref[co]
            h = jnp.maximum(h, _LEAK * h)
            w0 = wdb_ref[0, co, pl.ds(rp, 1), :, :]
            w1 = wdb_ref[1, co, pl.ds(rp, 1), :, :]
            oa0 = oa0 + jnp.sum(w0 * h, axis=(0, 1))
            oa1 = oa1 + jnp.sum(w1 * h, axis=(0, 1))
        return oa0, oa1
    oa0 = jnp.full((_BL,), bd_ref[0], jnp.float32)
    oa1 = jnp.full((_BL,), bd_ref[1], jnp.float32)
    oa0, oa1 = jax.lax.fori_loop(0, 18, c2_body, (oa0, oa1))
    out_ref[...] = jnp.stack([oa0, oa1])


def kernel(x, w1k, b1, w2k, b2, wd4, bd, rd1, cd1, rd2, cd2):
    del rd1, cd1, rd2, cd2
    x = x.reshape(-1, _H1, _W1).astype(jnp.float32)
    B = x.shape[0]
    Bp = ((B + _BL - 1) // _BL) * _BL
    xp = jnp.pad(x, ((0, Bp - B), (1, 1), (1, _WP - _W1 - 1)))
    xt = jnp.transpose(xp, (1, 2, 0))
    wdb = jnp.broadcast_to(wd4[..., None],
                           (_NCLS, _C2, _H3, _W3, _BL)).astype(jnp.float32)

    def _smem():
        return pl.BlockSpec(memory_space=pltpu.MemorySpace.SMEM)

    out = pl.pallas_call(
        _fwd_kernel,
        out_shape=jax.ShapeDtypeStruct((_NCLS, Bp), jnp.float32),
        grid=(Bp // _BL,),
        in_specs=[
            pl.BlockSpec((_HP, _WP, _BL), lambda b: (0, 0, b)),
            _smem(),
            _smem(),
            _smem(),
            _smem(),
            pl.BlockSpec((_NCLS, _C2, _H3, _W3, _BL),
                         lambda b: (0, 0, 0, 0, 0)),
            _smem(),
        ],
        out_specs=pl.BlockSpec((_NCLS, _BL), lambda b: (0, b)),
        scratch_shapes=[
            pltpu.VMEM((_C1, _P1H, _P1W, _BL), jnp.float32),
            pltpu.VMEM((_C1, _P1H, _P1W, _BL), jnp.float32),
            pltpu.VMEM((3, _HP, _WC, _BL), jnp.float32),
            pltpu.VMEM((1, _WC, _BL), jnp.float32),
        ],
        compiler_params=pltpu.CompilerParams(
            dimension_semantics=("arbitrary",)),
    )(xt, w1k, b1, w2k, b2, wdb, bd)
    return jnp.transpose(out)[:B]


# speedup vs baseline: 1.0717x; 1.0511x over previous
import jax
import jax.numpy as jnp
from jax.experimental import pallas as pl
from jax.experimental.pallas import tpu as pltpu

_H1, _W1 = 75, 69
_HP, _WP = 77, 80
_WC = 72
_P1H, _P1W = 39, 40
_WC2 = 36
_C1, _C2, _NCLS = 16, 4, 2
_H3, _W3 = 18, 17
_LEAK = 0.01
_BL = 128


def _fwd_kernel(x_ref, w1_ref, b1_ref, w2_ref, b2_ref, wdb_ref, bd_ref,
                out_ref, p1a_ref, p1b_ref, xs_ref, pwb_ref):
    @pl.when(pl.program_id(0) == 0)
    def _init():
        p1a_ref[...] = jnp.zeros_like(p1a_ref)
        p1b_ref[...] = jnp.zeros_like(p1b_ref)

    for kx in range(3):
        xs_ref[kx] = x_ref[:, kx:kx + _WC, :]

    def pool_cols(mv, w, win):
        pwb_ref[0:1, 0:w, :] = mv
        ev = pwb_ref[0:1, pl.ds(0, win // 2, 2), :]
        od = pwb_ref[0:1, pl.ds(1, win // 2, 2), :]
        return jnp.maximum(ev, od)

    def c1_body(rp, carry):
        r0 = 2 * rp
        for g in range(_C1 // 2):
            c0 = g * 2
            accs = [jnp.zeros((2, _WC, _BL), jnp.float32) for _ in range(2)]
            for ky in range(3):
                for kx in range(3):
                    k = ky * 3 + kx
                    p = xs_ref[kx, pl.ds(r0 + ky, 2), :, :]
                    for cc in range(2):
                        accs[cc] = accs[cc] + w1_ref[k * _C1 + c0 + cc] * p
            for cc, a in enumerate(accs):
                mv = jnp.maximum(a[0:1], a[1:2])
                pw = pool_cols(mv, _WC, 68)
                h = pw + b1_ref[c0 + cc]
                h = jnp.maximum(h, _LEAK * h)
                p1a_ref[c0 + cc, pl.ds(1 + rp, 1), 1:35, :] = h
                p1b_ref[c0 + cc, pl.ds(1 + rp, 1), 0:34, :] = h
        return carry
    jax.lax.fori_loop(0, 37, c1_body, 0, unroll=2)

    def c2_body(rp, oacc):
        oa0, oa1 = oacc
        z = jnp.zeros((2, _WC2, _BL), jnp.float32)
        accs = [z, z, z, z]
        for ci in range(_C1):
            for ky in range(3):
                for kx in range(3):
                    k = ky * 3 + kx
                    if kx == 0:
                        p = p1a_ref[ci, pl.ds(2 * rp + ky, 2), 0:_WC2, :]
                    elif kx == 1:
                        p = p1b_ref[ci, pl.ds(2 * rp + ky, 2), 0:_WC2, :]
                    else:
                        p = p1a_ref[ci, pl.ds(2 * rp + ky, 2), 2:2 + _WC2, :]
                    for co in range(_C2):
                        accs[co] = (accs[co]
                                    + w2_ref[(k * _C2 + co) * _C1 + ci] * p)
        for co in range(_C2):
            mv = jnp.maximum(accs[co][0:1], accs[co][1:2])
            pw = pool_cols(mv, _WC2, 34)
            h = pw + b2_ref[co]
            h = jnp.maximum(h, _LEAK * h)
            w0 = wdb_ref[0, co, pl.ds(rp, 1), :, :]
            w1 = wdb_ref[1, co, pl.ds(rp, 1), :, :]
            oa0 = oa0 + jnp.sum(w0 * h, axis=(0, 1))
            oa1 = oa1 + jnp.sum(w1 * h, axis=(0, 1))
        return oa0, oa1
    oa0 = jnp.full((_BL,), bd_ref[0], jnp.float32)
    oa1 = jnp.full((_BL,), bd_ref[1], jnp.float32)
    oa0, oa1 = jax.lax.fori_loop(0, 18, c2_body, (oa0, oa1), unroll=2)
    out_ref[...] = jnp.stack([oa0, oa1])


def kernel(x, w1k, b1, w2k, b2, wd4, bd, rd1, cd1, rd2, cd2):
    del rd1, cd1, rd2, cd2
    x = x.reshape(-1, _H1, _W1).astype(jnp.float32)
    B = x.shape[0]
    Bp = ((B + _BL - 1) // _BL) * _BL
    xp = jnp.pad(x, ((0, Bp - B), (1, 1), (1, _WP - _W1 - 1)))
    xt = jnp.transpose(xp, (1, 2, 0))
    wdb = jnp.broadcast_to(wd4[..., None],
                           (_NCLS, _C2, _H3, _W3, _BL)).astype(jnp.float32)

    def _smem():
        return pl.BlockSpec(memory_space=pltpu.MemorySpace.SMEM)

    out = pl.pallas_call(
        _fwd_kernel,
        out_shape=jax.ShapeDtypeStruct((_NCLS, Bp), jnp.float32),
        grid=(Bp // _BL,),
        in_specs=[
            pl.BlockSpec((_HP, _WP, _BL), lambda b: (0, 0, b)),
            _smem(),
            _smem(),
            _smem(),
            _smem(),
            pl.BlockSpec((_NCLS, _C2, _H3, _W3, _BL),
                         lambda b: (0, 0, 0, 0, 0)),
            _smem(),
        ],
        out_specs=pl.BlockSpec((_NCLS, _BL), lambda b: (0, b)),
        scratch_shapes=[
            pltpu.VMEM((_C1, _P1H, _P1W, _BL), jnp.float32),
            pltpu.VMEM((_C1, _P1H, _P1W, _BL), jnp.float32),
            pltpu.VMEM((3, _HP, _WC, _BL), jnp.float32),
            pltpu.VMEM((1, _WC, _BL), jnp.float32),
        ],
        compiler_params=pltpu.CompilerParams(
            dimension_semantics=("arbitrary",)),
    )(xt, w1k, b1, w2k, b2, wdb, bd)
    return jnp.transpose(out)[:B]


# speedup vs baseline: 1.0762x; 1.0042x over previous
import jax
import jax.numpy as jnp
from jax.experimental import pallas as pl
from jax.experimental.pallas import tpu as pltpu

_H1, _W1 = 75, 69
_HP, _WP = 77, 80
_WC = 72
_P1H, _P1W = 39, 40
_WC2 = 36
_C1, _C2, _NCLS = 16, 4, 2
_H3, _W3 = 18, 17
_LEAK = 0.01
_BL = 128


def _fwd_kernel(x_ref, w1_ref, b1_ref, w2_ref, b2_ref, wdb_ref, bd_ref,
                out_ref, p1a_ref, p1b_ref, xs_ref, pwb_ref):
    @pl.when(pl.program_id(0) == 0)
    def _init():
        p1a_ref[...] = jnp.zeros_like(p1a_ref)
        p1b_ref[...] = jnp.zeros_like(p1b_ref)

    for kx in range(3):
        xs_ref[kx] = x_ref[:, kx:kx + _WC, :]

    def pool_cols(mv, w, win):
        pwb_ref[0:1, 0:w, :] = mv
        ev = pwb_ref[0:1, pl.ds(0, win // 2, 2), :]
        od = pwb_ref[0:1, pl.ds(1, win // 2, 2), :]
        return jnp.maximum(ev, od)

    def c1_body(rp, carry):
        r0 = 2 * rp
        for g in range(_C1 // 2):
            c0 = g * 2
            accs = [jnp.zeros((2, _WC, _BL), jnp.float32) for _ in range(2)]
            for ky in range(3):
                for kx in range(3):
                    k = ky * 3 + kx
                    p = xs_ref[kx, pl.ds(r0 + ky, 2), :, :]
                    for cc in range(2):
                        accs[cc] = accs[cc] + w1_ref[k * _C1 + c0 + cc] * p
            for cc, a in enumerate(accs):
                mv = jnp.maximum(a[0:1], a[1:2])
                pw = pool_cols(mv, _WC, 68)
                h = pw + b1_ref[c0 + cc]
                h = jnp.maximum(h, _LEAK * h)
                p1a_ref[c0 + cc, pl.ds(1 + rp, 1), 1:35, :] = h
                p1b_ref[c0 + cc, pl.ds(1 + rp, 1), 0:34, :] = h
        return carry
    jax.lax.fori_loop(0, 37, c1_body, 0, unroll=4)

    def c2_body(rp, oacc):
        oa0, oa1 = oacc
        z = jnp.zeros((2, _WC2, _BL), jnp.float32)
        accs = [z, z, z, z]
        for ci in range(_C1):
            for ky in range(3):
                for kx in range(3):
                    k = ky * 3 + kx
                    if kx == 0:
                        p = p1a_ref[ci, pl.ds(2 * rp + ky, 2), 0:_WC2, :]
                    elif kx == 1:
                        p = p1b_ref[ci, pl.ds(2 * rp + ky, 2), 0:_WC2, :]
                    else:
                        p = p1a_ref[ci, pl.ds(2 * rp + ky, 2), 2:2 + _WC2, :]
                    for co in range(_C2):
                        accs[co] = (accs[co]
                                    + w2_ref[(k * _C2 + co) * _C1 + ci] * p)
        for co in range(_C2):
            mv = jnp.maximum(accs[co][0:1], accs[co][1:2])
            pw = pool_cols(mv, _WC2, 34)
            h = pw + b2_ref[co]
            h = jnp.maximum(h, _LEAK * h)
            w0 = wdb_ref[0, co, pl.ds(rp, 1), :, :]
            w1 = wdb_ref[1, co, pl.ds(rp, 1), :, :]
            oa0 = oa0 + jnp.sum(w0 * h, axis=(0, 1))
            oa1 = oa1 + jnp.sum(w1 * h, axis=(0, 1))
        return oa0, oa1
    oa0 = jnp.full((_BL,), bd_ref[0], jnp.float32)
    oa1 = jnp.full((_BL,), bd_ref[1], jnp.float32)
    oa0, oa1 = jax.lax.fori_loop(0, 18, c2_body, (oa0, oa1), unroll=4)
    out_ref[...] = jnp.stack([oa0, oa1])


def kernel(x, w1k, b1, w2k, b2, wd4, bd, rd1, cd1, rd2, cd2):
    del rd1, cd1, rd2, cd2
    x = x.reshape(-1, _H1, _W1).astype(jnp.float32)
    B = x.shape[0]
    Bp = ((B + _BL - 1) // _BL) * _BL
    xp = jnp.pad(x, ((0, Bp - B), (1, 1), (1, _WP - _W1 - 1)))
    xt = jnp.transpose(xp, (1, 2, 0))
    wdb = jnp.broadcast_to(wd4[..., None],
                           (_NCLS, _C2, _H3, _W3, _BL)).astype(jnp.float32)

    def _smem():
        return pl.BlockSpec(memory_space=pltpu.MemorySpace.SMEM)

    out = pl.pallas_call(
        _fwd_kernel,
        out_shape=jax.ShapeDtypeStruct((_NCLS, Bp), jnp.float32),
        grid=(Bp // _BL,),
        in_specs=[
            pl.BlockSpec((_HP, _WP, _BL), lambda b: (0, 0, b)),
            _smem(),
            _smem(),
            _smem(),
            _smem(),
            pl.BlockSpec((_NCLS, _C2, _H3, _W3, _BL),
                         lambda b: (0, 0, 0, 0, 0)),
            _smem(),
        ],
        out_specs=pl.BlockSpec((_NCLS, _BL), lambda b: (0, b)),
        scratch_shapes=[
            pltpu.VMEM((_C1, _P1H, _P1W, _BL), jnp.float32),
            pltpu.VMEM((_C1, _P1H, _P1W, _BL), jnp.float32),
            pltpu.VMEM((3, _HP, _WC, _BL), jnp.float32),
            pltpu.VMEM((1, _WC, _BL), jnp.float32),
        ],
        compiler_params=pltpu.CompilerParams(
            dimension_semantics=("arbitrary",)),
    )(xt, w1k, b1, w2k, b2, wdb, bd)
    return jnp.transpose(out)[:B]
